# baseline (speedup 1.0000x reference)
import functools

import jax
import jax.numpy as jnp
from jax import lax
from jax.experimental import pallas as pl
from jax.experimental.pallas import tpu as pltpu
from jax.experimental.pallas import tpu_sc as plsc

POOL = 7
SAMP = 2
SCALE = 0.25
N, C, H, W = 4, 256, 56, 56
NROWS = N * H * W
NR = 1000
RPAD = 1024
NWORK = 32
RPW = RPAD // NWORK
NBINS = POOL * POOL
NBUF = 8
L = 16


def _roi_align_kernel(feat_hbm, rois_hbm, out_hbm, roi_v, yi_t, wy_t, xi_t,
                      wx_t, wtab, itab, g_b, stage, *sems):
    wid = lax.axis_index("s") * 2 + lax.axis_index("c")
    pltpu.sync_copy(rois_hbm.at[pl.ds(wid * (RPW * 8), RPW * 8)],
                    roi_v.at[pl.ds(0, RPW * 8)])

    iota = lax.broadcasted_iota(jnp.int32, (L,), 0)
    dy = (iota >> 3) & 1
    cy = (iota >> 2) & 1
    dx = (iota >> 1) & 1
    cx = iota & 1
    kyc = cy * L + dy
    kxc = cx * L + dx
    tvec = (iota.astype(jnp.float32) + 0.5) * (1.0 / SAMP)

    def bcast_roi(r, j):
        return plsc.load_gather(roi_v, [iota * 0 + (r * 8 + j)])

    def axis_tables(lo_v, hi_v, bound, ii_t, ww_t):
        span = jnp.maximum(hi_v - lo_v, 1.0)
        binsz = span / POOL
        vv = lo_v + tvec * binsz
        vf = jnp.where((vv > -1.0) & (vv < float(bound)), 1.0, 0.0)
        vcl = jnp.clip(vv, 0.0, float(bound) - 1.0)
        i0 = vcl.astype(jnp.int32)
        i1 = jnp.minimum(i0 + 1, int(bound) - 1)
        frac = vcl - i0.astype(jnp.float32)
        ii_t[pl.ds(0, L)] = i0
        ii_t[pl.ds(L, L)] = i1
        ww_t[pl.ds(0, L)] = (1.0 - frac) * vf
        ww_t[pl.ds(L, L)] = frac * vf

    def build_tables(binid, carry):
        ph = binid // POOL
        pw = binid % POOL
        iy = kyc + 2 * ph
        ix = kxc + 2 * pw
        yi = plsc.load_gather(yi_t, [iy])
        wy = plsc.load_gather(wy_t, [iy])
        xi = plsc.load_gather(xi_t, [ix])
        wx = plsc.load_gather(wx_t, [ix])
        itab[binid, pl.ds(0, L)] = yi * W + xi
        wtab[pl.ds(binid * L, L)] = wy * wx * 0.25
        return carry

    def fire(binid, buf, sem):
        pltpu.async_copy(feat_hbm.at[itab.at[binid]], g_b.at[buf], sem)

    def wait(binid, buf, sem):
        pltpu.make_async_copy(feat_hbm.at[itab.at[binid]], g_b.at[buf],
                              sem).wait()

    def accum(binid, buf):
        wk = [plsc.load_gather(wtab, [iota * 0 + (binid * L + k)])
              for k in range(L)]
        himask = jnp.full((L,), -65536, jnp.int32)
        for j in range(C // 32):
            acc_lo = jnp.zeros((L,), jnp.float32)
            acc_hi = jnp.zeros((L,), jnp.float32)
            for k in range(L):
                ci = g_b[buf, k, pl.ds(j * L, L)]
                lo = plsc.bitcast(ci << 16, jnp.float32)
                hi = plsc.bitcast(ci & himask, jnp.float32)
                acc_lo = acc_lo + wk[k] * lo
                acc_hi = acc_hi + wk[k] * hi
            sidx = iota * (2 * NBINS) + (binid + j * 32 * NBINS)
            plsc.store_scatter(stage, [sidx], acc_lo)
            plsc.store_scatter(stage, [sidx + NBINS], acc_hi)

    def do_roi(r, carry):
        brow = bcast_roi(r, 0).astype(jnp.int32) * H
        x1 = bcast_roi(r, 1) * SCALE
        y1 = bcast_roi(r, 2) * SCALE
        x2 = bcast_roi(r, 3) * SCALE
        y2 = bcast_roi(r, 4) * SCALE
        axis_tables(y1, y2, H, yi_t, wy_t)
        axis_tables(x1, x2, W, xi_t, wx_t)
        yi_t[pl.ds(0, L)] = yi_t[pl.ds(0, L)] + brow
        yi_t[pl.ds(L, L)] = yi_t[pl.ds(L, L)] + brow

        lax.fori_loop(0, NBINS, build_tables, 0)

        for b in range(NBUF):
            fire(b, b, sems[b])

        def step(i, c):
            base = i * NBUF
            for b in range(NBUF):
                binid = base + b
                wait(binid, b, sems[b])
                accum(binid, b)

                @pl.when(binid + NBUF < NBINS)
                def _():
                    fire(binid + NBUF, b, sems[b])

            return c

        NSTEP = (NBINS - 1) // NBUF
        lax.fori_loop(0, NSTEP, step, 0)
        for b in range(NBUF):
            binid = NSTEP * NBUF + b
            if binid >= NBINS:
                break
            wait(binid, b, sems[b])
            accum(binid, b)
            if binid + NBUF < NBINS:
                fire(binid + NBUF, b, sems[b])
        rglob = wid * RPW + r

        @pl.when(rglob < NR)
        def _():
            pltpu.sync_copy(stage, out_hbm.at[rglob])

        return carry

    lax.fori_loop(0, RPW, do_roi, 0)


@jax.jit
def _roi_align_sc(feat_flat, rois_flat):
    mesh = plsc.VectorSubcoreMesh(core_axis_name="c", subcore_axis_name="s",
                                  num_cores=2, num_subcores=16)
    kern = functools.partial(
        pl.kernel,
        out_type=jax.ShapeDtypeStruct((NR, C * NBINS), jnp.float32),
        mesh=mesh,
        compiler_params=pltpu.CompilerParams(needs_layout_passes=False),
        scratch_types=[
            pltpu.VMEM((RPW * 8 + 8,), jnp.float32),
            pltpu.VMEM((2 * L,), jnp.int32),
            pltpu.VMEM((2 * L,), jnp.float32),
            pltpu.VMEM((2 * L,), jnp.int32),
            pltpu.VMEM((2 * L,), jnp.float32),
            pltpu.VMEM((NBINS * L,), jnp.float32),
            pltpu.VMEM((NBINS, L), jnp.int32),
            pltpu.VMEM((NBUF, L, C // 2), jnp.int32),
            pltpu.VMEM((C * NBINS,), jnp.float32),
        ] + [pltpu.SemaphoreType.DMA] * NBUF,
    )(_roi_align_kernel)
    return kern(feat_flat, rois_flat)


def kernel(feat, rois):
    feat_flat = feat.transpose(0, 2, 3, 1).reshape(NROWS, C)
    feat_i32 = jax.lax.bitcast_convert_type(
        feat_flat.astype(jnp.bfloat16).reshape(NROWS, C // 2, 2), jnp.int32)
    rois8 = jnp.zeros((RPAD, 8), jnp.float32).at[:rois.shape[0], :5].set(rois)
    out = _roi_align_sc(feat_i32, rois8.reshape(-1))
    return out.reshape(rois.shape[0], C, POOL, POOL)


# speedup vs baseline: 1.5812x; 1.5812x over previous
import functools

import jax
import jax.numpy as jnp
from jax import lax
from jax.experimental import pallas as pl
from jax.experimental.pallas import tpu as pltpu
from jax.experimental.pallas import tpu_sc as plsc

POOL = 7
SAMP = 2
SCALE = 0.25
N, C, H, W = 4, 256, 56, 56
NROWS = N * H * W
NR = 1000
RPAD = 1024
NWORK = 32
RPW = RPAD // NWORK
NBINS = POOL * POOL
NBUF = 4
L = 16


def _roi_align_kernel(feat_hbm, rois_hbm, out_hbm, roi_v, yi_t, wy_t, xi_t,
                      wx_t, wtab, itab, g_b, stage, *sems):
    wid = lax.axis_index("s") * 2 + lax.axis_index("c")
    pltpu.sync_copy(rois_hbm.at[pl.ds(wid * (RPW * 8), RPW * 8)],
                    roi_v.at[pl.ds(0, RPW * 8)])

    iota = lax.broadcasted_iota(jnp.int32, (L,), 0)
    dy = (iota >> 3) & 1
    cy = (iota >> 2) & 1
    dx = (iota >> 1) & 1
    cx = iota & 1
    kyc = cy * L + dy
    kxc = cx * L + dx
    tvec = (iota.astype(jnp.float32) + 0.5) * (1.0 / SAMP)

    def bcast_roi(r, j):
        return plsc.load_gather(roi_v, [iota * 0 + (r * 8 + j)])

    def axis_tables(lo_v, hi_v, bound, ii_t, ww_t):
        span = jnp.maximum(hi_v - lo_v, 1.0)
        binsz = span / POOL
        vv = lo_v + tvec * binsz
        vf = jnp.where((vv > -1.0) & (vv < float(bound)), 1.0, 0.0)
        vcl = jnp.clip(vv, 0.0, float(bound) - 1.0)
        i0 = vcl.astype(jnp.int32)
        i1 = jnp.minimum(i0 + 1, int(bound) - 1)
        frac = vcl - i0.astype(jnp.float32)
        ii_t[pl.ds(0, L)] = i0
        ii_t[pl.ds(L, L)] = i1
        ww_t[pl.ds(0, L)] = (1.0 - frac) * vf
        ww_t[pl.ds(L, L)] = frac * vf

    def build_tables(binid, carry):
        ph = binid // POOL
        pw = binid % POOL
        iy = kyc + 2 * ph
        ix = kxc + 2 * pw
        yi = plsc.load_gather(yi_t, [iy])
        wy = plsc.load_gather(wy_t, [iy])
        xi = plsc.load_gather(xi_t, [ix])
        wx = plsc.load_gather(wx_t, [ix])
        itab[binid, pl.ds(0, L)] = yi * W + xi
        wtab[pl.ds(binid * L, L)] = wy * wx * 0.25
        return carry

    def fire(binid, buf, sem):
        pltpu.async_copy(feat_hbm.at[itab.at[binid]], g_b.at[buf], sem)

    def wait(binid, buf, sem):
        pltpu.make_async_copy(feat_hbm.at[itab.at[binid]], g_b.at[buf],
                              sem).wait()

    def accum(binid, buf):
        wk = [plsc.load_gather(wtab, [iota * 0 + (binid * L + k)])
              for k in range(L)]
        himask = jnp.full((L,), -65536, jnp.int32)
        for j in range(C // 32):
            acc_lo = jnp.zeros((L,), jnp.float32)
            acc_hi = jnp.zeros((L,), jnp.float32)
            for k in range(L):
                ci = g_b[buf, k, pl.ds(j * L, L)]
                lo = plsc.bitcast(ci << 16, jnp.float32)
                hi = plsc.bitcast(ci & himask, jnp.float32)
                acc_lo = acc_lo + wk[k] * lo
                acc_hi = acc_hi + wk[k] * hi
            sidx = iota * (2 * NBINS) + (binid + j * 32 * NBINS)
            plsc.store_scatter(stage, [sidx], acc_lo)
            plsc.store_scatter(stage, [sidx + NBINS], acc_hi)

    def do_roi(r, carry):
        brow = bcast_roi(r, 0).astype(jnp.int32) * H
        x1 = bcast_roi(r, 1) * SCALE
        y1 = bcast_roi(r, 2) * SCALE
        x2 = bcast_roi(r, 3) * SCALE
        y2 = bcast_roi(r, 4) * SCALE
        axis_tables(y1, y2, H, yi_t, wy_t)
        axis_tables(x1, x2, W, xi_t, wx_t)
        yi_t[pl.ds(0, L)] = yi_t[pl.ds(0, L)] + brow
        yi_t[pl.ds(L, L)] = yi_t[pl.ds(L, L)] + brow

        lax.fori_loop(0, NBINS, build_tables, 0)

        for b in range(NBUF):
            fire(b, b, sems[b])

        def step(i, c):
            base = i * NBUF
            for b in range(NBUF):
                binid = base + b
                wait(binid, b, sems[b])
                accum(binid, b)

                @pl.when(binid + NBUF < NBINS)
                def _():
                    fire(binid + NBUF, b, sems[b])

            return c

        NSTEP = (NBINS - 1) // NBUF
        lax.fori_loop(0, NSTEP, step, 0)
        for b in range(NBUF):
            binid = NSTEP * NBUF + b
            if binid >= NBINS:
                break
            wait(binid, b, sems[b])
            accum(binid, b)
            if binid + NBUF < NBINS:
                fire(binid + NBUF, b, sems[b])
        rglob = wid * RPW + r

        @pl.when(rglob < NR)
        def _():
            pltpu.sync_copy(stage, out_hbm.at[rglob])

        return carry

    lax.fori_loop(0, RPW, do_roi, 0)


@jax.jit
def _roi_align_sc(feat_flat, rois_flat):
    mesh = plsc.VectorSubcoreMesh(core_axis_name="c", subcore_axis_name="s",
                                  num_cores=2, num_subcores=16)
    kern = functools.partial(
        pl.kernel,
        out_type=jax.ShapeDtypeStruct((NR, C * NBINS), jnp.float32),
        mesh=mesh,
        compiler_params=pltpu.CompilerParams(needs_layout_passes=False),
        scratch_types=[
            pltpu.VMEM((RPW * 8 + 8,), jnp.float32),
            pltpu.VMEM((2 * L,), jnp.int32),
            pltpu.VMEM((2 * L,), jnp.float32),
            pltpu.VMEM((2 * L,), jnp.int32),
            pltpu.VMEM((2 * L,), jnp.float32),
            pltpu.VMEM((NBINS * L,), jnp.float32),
            pltpu.VMEM((NBINS, L), jnp.int32),
            pltpu.VMEM((NBUF, L, C // 2), jnp.int32),
            pltpu.VMEM((C * NBINS,), jnp.float32),
        ] + [pltpu.SemaphoreType.DMA] * NBUF,
    )(_roi_align_kernel)
    return kern(feat_flat, rois_flat)


def kernel(feat, rois):
    feat_flat = feat.transpose(0, 2, 3, 1).reshape(NROWS, C)
    feat_i32 = jax.lax.bitcast_convert_type(
        feat_flat.astype(jnp.bfloat16).reshape(NROWS, C // 2, 2), jnp.int32)
    rois8 = jnp.zeros((RPAD, 8), jnp.float32).at[:rois.shape[0], :5].set(rois)
    out = _roi_align_sc(feat_i32, rois8.reshape(-1))
    return out.reshape(rois.shape[0], C, POOL, POOL)
